# Initial kernel scaffold; baseline (speedup 1.0000x reference)
import jax
import jax.numpy as jnp
from jax.experimental import pallas as pl


def kernel(feat, edge_index, weight, bias, linear_comb):
    raise NotImplementedError("write your pallas kernel here")


# speedup vs baseline: 4.1995x; 4.1995x over previous
import functools

import jax
import jax.numpy as jnp
from jax import lax
from jax.experimental import pallas as pl
from jax.experimental.pallas import tpu as pltpu
from jax.experimental.pallas import tpu_sc as plsc

N = 10000
D = 128
E = 320000

NC = 2
NS = 16
NW = NC * NS
K = 128
EPT = -(-E // (NW * K)) * K
EPAD = EPT * NW
NIT = EPT // K
NP = 10240
RPT = NP // NS
DW = 16

R = 2000
NBLK = N // R


def _sc_mesh():
    return plsc.VectorSubcoreMesh(core_axis_name="c", subcore_axis_name="s")


@functools.partial(
    pl.kernel,
    mesh=_sc_mesh(),
    out_type=jax.ShapeDtypeStruct((2, NP), jnp.float32),
    scratch_types=[
        pltpu.VMEM((K,), jnp.float32),
        pltpu.VMEM((RPT,), jnp.float32),
        pltpu.VMEM((K,), jnp.int32),
        pltpu.VMEM_SHARED((NP,), jnp.float32),
    ],
)
def _deg_kernel(dst_hbm, out_hbm, ones_v, zero_v, idx_v, acc_sh):
    cid = lax.axis_index("c")
    sid = lax.axis_index("s")
    wid = cid * NS + sid
    for j in range(K // 16):
        ones_v[pl.ds(j * 16, 16)] = jnp.full((16,), 1.0, jnp.float32)
    for j in range(RPT // 16):
        zero_v[pl.ds(j * 16, 16)] = jnp.zeros((16,), jnp.float32)
    pltpu.sync_copy(zero_v, acc_sh.at[pl.ds(sid * RPT, RPT)])
    plsc.subcore_barrier()

    def body(g, carry):
        base = pl.multiple_of(wid * EPT + g * K, 8)
        pltpu.sync_copy(dst_hbm.at[pl.ds(base, K)], idx_v)
        pltpu.sync_copy(ones_v, acc_sh.at[idx_v], add=True)
        return carry

    lax.fori_loop(0, NIT, body, 0)
    plsc.subcore_barrier()
    pltpu.sync_copy(acc_sh.at[pl.ds(sid * RPT, RPT)],
                    out_hbm.at[cid, pl.ds(sid * RPT, RPT)])


@functools.partial(
    pl.kernel,
    mesh=_sc_mesh(),
    out_type=jax.ShapeDtypeStruct((2, NP, D), jnp.float32),
    scratch_types=[
        pltpu.VMEM((K, D), jnp.float32),
        pltpu.VMEM((K,), jnp.int32),
        pltpu.VMEM((K,), jnp.int32),
        pltpu.VMEM_SHARED((NP, D), jnp.float32),
        pltpu.SemaphoreType.DMA,
    ],
)
def _msg_kernel(src_hbm, dst_hbm, h_hbm, zeros_hbm, out_hbm,
                rows_v, isrc_v, idst_v, acc_sh, sem):
    cid = lax.axis_index("c")
    sid = lax.axis_index("s")
    wid = cid * NS + sid
    pltpu.sync_copy(zeros_hbm, acc_sh.at[pl.ds(sid * RPT, RPT)])
    plsc.subcore_barrier()

    def body(g, carry):
        base = pl.multiple_of(wid * EPT + g * K, 8)
        pltpu.sync_copy(src_hbm.at[pl.ds(base, K)], isrc_v)
        pltpu.sync_copy(dst_hbm.at[pl.ds(base, K)], idst_v)
        pltpu.async_copy(h_hbm.at[isrc_v], rows_v, sem).wait()
        pltpu.sync_copy(rows_v, acc_sh.at[idst_v], add=True)
        return carry

    lax.fori_loop(0, NIT, body, 0)
    plsc.subcore_barrier()
    pltpu.sync_copy(acc_sh.at[pl.ds(sid * RPT, RPT)],
                    out_hbm.at[cid, pl.ds(sid * RPT, RPT)])


def _h_body(feat_ref, dga_ref, dgb_ref, h_ref):
    d = dga_ref[0] + dgb_ref[0]
    norm = lax.rsqrt(jnp.maximum(d, 1.0))
    h_ref[...] = feat_ref[...] * norm


def _h_call(feat, deg2):
    return pl.pallas_call(
        _h_body,
        grid=(NBLK,),
        in_specs=[
            pl.BlockSpec((R, D), lambda i: (i, 0)),
            pl.BlockSpec((1, R, 1), lambda i: (0, i, 0)),
            pl.BlockSpec((1, R, 1), lambda i: (1, i, 0)),
        ],
        out_specs=pl.BlockSpec((R, D), lambda i: (i, 0)),
        out_shape=jax.ShapeDtypeStruct((N, D), jnp.float32),
    )(feat, deg2, deg2)


def _out_body(msga_ref, msgb_ref, h_ref, dga_ref, dgb_ref, lin_ref,
              w_ref, b_ref, o_ref):
    msg = msga_ref[0] + msgb_ref[0]
    d = dga_ref[0] + dgb_ref[0]
    norm = lax.rsqrt(jnp.maximum(d, 1.0))
    l = lin_ref[...]
    out = (1.0 - l) * msg + l * h_ref[...]
    r = jnp.dot(out, w_ref[...], preferred_element_type=jnp.float32)
    o_ref[...] = r * norm + b_ref[...]


def _out_call(msg2, h, deg2, lin, weight, bias):
    return pl.pallas_call(
        _out_body,
        grid=(NBLK,),
        in_specs=[
            pl.BlockSpec((1, R, D), lambda i: (0, i, 0)),
            pl.BlockSpec((1, R, D), lambda i: (1, i, 0)),
            pl.BlockSpec((R, D), lambda i: (i, 0)),
            pl.BlockSpec((1, R, 1), lambda i: (0, i, 0)),
            pl.BlockSpec((1, R, 1), lambda i: (1, i, 0)),
            pl.BlockSpec((R, 1), lambda i: (i, 0)),
            pl.BlockSpec((D, D), lambda i: (0, 0)),
            pl.BlockSpec((1, D), lambda i: (0, 0)),
        ],
        out_specs=pl.BlockSpec((R, D), lambda i: (i, 0)),
        out_shape=jax.ShapeDtypeStruct((N, D), jnp.float32),
    )(msg2, msg2, h, deg2, deg2, lin, weight, bias)


def kernel(feat, edge_index, weight, bias, linear_comb):
    pad = EPAD - E
    src = jnp.concatenate([edge_index[0], jnp.zeros((pad,), jnp.int32)])
    dst = jnp.concatenate([edge_index[1], jnp.full((pad,), N, jnp.int32)])
    zeros128 = jnp.zeros((RPT, D), jnp.float32)

    deg2 = _deg_kernel(dst).reshape(2, NP, 1)
    h = _h_call(feat, deg2)
    msg2 = _msg_kernel(src, dst, h, zeros128)
    rst = _out_call(msg2, h, deg2, linear_comb.reshape(N, 1),
                    weight, bias.reshape(1, D))
    return rst
